# baseline (speedup 1.0000x reference)
import functools
import math

import jax
import jax.numpy as jnp
from jax import lax
from jax.experimental import pallas as pl
from jax.experimental.pallas import tpu as pltpu
from jax.experimental.pallas import tpu_sc as plsc

VOCAB = 100000
EMB = 64
N_BASE = 16384
N_POS = 8192
N_OUT = N_BASE + N_POS

NC, NS, L = 2, 16, 16
NW = NC * NS
BPW = N_OUT // NW
GCH = 128
NCHUNK = BPW // GCH


def _posenc_body(pt_ref, out_ref):
    pt = pt_ref[...].astype(jnp.float32)
    coli = lax.broadcasted_iota(jnp.int32, (N_POS, EMB), 1)
    col = coli.astype(jnp.float32)
    half = jnp.where(coli < EMB // 2, col, col - EMB // 2)
    period = jnp.exp(half * (-2.0 * math.log(10000.0) / EMB))
    arg = pt * period
    out_ref[...] = jnp.where(coli < EMB // 2, jnp.sin(arg), jnp.cos(arg))


_posenc = pl.pallas_call(
    _posenc_body,
    out_shape=jax.ShapeDtypeStruct((N_POS, EMB), jnp.float32),
)


def _sc_body(bmt_hbm, idx_hbm, pe_hbm, table_hbm, out_hbm,
             bmt_v, idx_v, ti_v, pi_v, m_v, rt_v, rp_v, sem):
    wid = lax.axis_index("s") * NC + lax.axis_index("c")
    base = wid * BPW
    pltpu.sync_copy(bmt_hbm, bmt_v)
    pltpu.sync_copy(idx_hbm.at[pl.ds(base, BPW)], idx_v)

    for k in range(BPW // L):
        sl = pl.ds(k * L, L)
        iv = idx_v[sl]
        isb = iv < N_BASE
        tok = plsc.load_gather(bmt_v, [lax.rem(iv, N_BASE)])
        ti_v[k // (GCH // L), pl.ds((k % (GCH // L)) * L, L)] = tok
        pi_v[k // (GCH // L), pl.ds((k % (GCH // L)) * L, L)] = (
            lax.rem(iv, N_POS))
        m_v[sl] = jnp.where(isb, 1.0, 0.0).astype(jnp.float32)

    copies = []
    for j in range(NCHUNK):
        copies.append(pltpu.async_copy(
            table_hbm.at[ti_v.at[j]], rt_v.at[pl.ds(j * GCH, GCH)], sem))
    for j in range(NCHUNK):
        copies.append(pltpu.async_copy(
            pe_hbm.at[pi_v.at[j]], rp_v.at[pl.ds(j * GCH, GCH)], sem))
    for c in copies:
        c.wait()

    def mrow(r, carry):
        mv = plsc.load_gather(m_v, [jnp.full((L,), 0, jnp.int32) + r])
        keep = mv > 0.5
        for q in range(EMB // L):
            csl = pl.ds(q * L, L)
            rt_v[r, csl] = jnp.where(keep, rt_v[r, csl], rp_v[r, csl])
        return carry
    lax.fori_loop(0, BPW, mrow, 0)

    pltpu.sync_copy(rt_v, out_hbm.at[pl.ds(base, BPW)])


_sc_call = functools.partial(
    pl.kernel,
    out_type=jax.ShapeDtypeStruct((N_OUT, EMB), jnp.float32),
    mesh=plsc.VectorSubcoreMesh(core_axis_name="c", subcore_axis_name="s"),
    compiler_params=pltpu.CompilerParams(needs_layout_passes=False,
                                         use_tc_tiling_on_sc=False),
    scratch_types=[
        pltpu.VMEM((N_BASE,), jnp.int32),
        pltpu.VMEM((BPW,), jnp.int32),
        pltpu.VMEM((NCHUNK, GCH), jnp.int32),
        pltpu.VMEM((NCHUNK, GCH), jnp.int32),
        pltpu.VMEM((BPW,), jnp.float32),
        pltpu.VMEM((BPW, EMB), jnp.float32),
        pltpu.VMEM((BPW, EMB), jnp.float32),
        pltpu.SemaphoreType.DMA,
    ],
)(_sc_body)


def kernel(base_model_tokens, positional_tokens, base_idxs_of_tokens, table):
    pe = _posenc(positional_tokens.astype(jnp.int32).reshape(N_POS, 1))
    return _sc_call(base_model_tokens.astype(jnp.int32),
                    base_idxs_of_tokens.astype(jnp.int32), pe, table)


# speedup vs baseline: 1.1470x; 1.1470x over previous
import functools
import math

import jax
import jax.numpy as jnp
from jax import lax
from jax.experimental import pallas as pl
from jax.experimental.pallas import tpu as pltpu
from jax.experimental.pallas import tpu_sc as plsc

VOCAB = 100000
EMB = 64
N_BASE = 16384
N_POS = 8192
N_OUT = N_BASE + N_POS

NC, NS, L = 2, 16, 16
NW = NC * NS
CPW = EMB // NW
CHUNK = 1536
NCHUNK = N_OUT // CHUNK
LT = 128
VFULL = VOCAB // LT
VPAD = (VFULL + 1) * LT
COLS = VPAD + N_POS


def _posenc_body(pt_ref, out_ref):
    pt = pt_ref[...].astype(jnp.float32)[None, :]
    row = lax.broadcasted_iota(jnp.int32, (EMB, 1), 0)
    k = (row % (EMB // 2)).astype(jnp.float32)
    period = jnp.exp(k * (-2.0 * math.log(10000.0) / EMB))
    shift = jnp.where(row < EMB // 2, 0.0, 0.5 * math.pi)
    out_ref[...] = jnp.sin(pt * period + shift)


_posenc = pl.pallas_call(
    _posenc_body,
    out_shape=jax.ShapeDtypeStruct((EMB, N_POS), jnp.float32),
)


def _sc_body(bmt_hbm, idx_hbm, pe_hbm, tt_hbm, tail_hbm, out_hbm,
             bmt_v, col_v, idx_v, out_v, lsem, wsem):
    wid = lax.axis_index("s") * NC + lax.axis_index("c")
    pltpu.sync_copy(bmt_hbm, bmt_v)

    def do_column(cix):
        def fire_tt(g, carry):
            pltpu.async_copy(tt_hbm.at[cix, pl.ds(g * LT, LT)],
                             col_v.at[pl.ds(g * LT, LT)], lsem)
            return carry
        lax.fori_loop(0, VFULL, fire_tt, 0)
        pltpu.async_copy(tail_hbm.at[cix], col_v.at[pl.ds(VFULL * LT, LT)],
                         lsem)
        def fire_pe(g, carry):
            pltpu.async_copy(pe_hbm.at[cix, pl.ds(g * LT, LT)],
                             col_v.at[pl.ds(VPAD + g * LT, LT)], lsem)
            return carry
        lax.fori_loop(0, N_POS // LT, fire_pe, 0)
        def wait_col(g, carry):
            pltpu.make_async_copy(tt_hbm.at[cix, pl.ds(0, LT)],
                                  col_v.at[pl.ds(0, LT)], lsem).wait()
            return carry
        lax.fori_loop(0, VFULL + 1 + N_POS // LT, wait_col, 0)

        for ch in range(NCHUNK):
            pltpu.sync_copy(idx_hbm.at[pl.ds(ch * CHUNK, CHUNK)], idx_v)
            if ch > 0:
                def wait_out(g, carry):
                    pltpu.make_async_copy(
                        out_v.at[pl.ds(0, LT)],
                        out_hbm.at[cix, pl.ds(0, LT)], wsem).wait()
                    return carry
                lax.fori_loop(0, CHUNK // LT, wait_out, 0)

            def grp(g, carry):
                sl = pl.ds(g * L, L)
                iv = idx_v[sl]
                isb = iv < N_BASE
                tok = plsc.load_gather(bmt_v, [lax.rem(iv, N_BASE)])
                comb = jnp.where(isb, tok, iv + (VPAD - N_BASE))
                out_v[sl] = plsc.load_gather(col_v, [comb])
                return carry
            lax.fori_loop(0, CHUNK // L, grp, 0)

            def fire_out(g, carry):
                pltpu.async_copy(
                    out_v.at[pl.ds(g * LT, LT)],
                    out_hbm.at[cix, pl.ds(ch * CHUNK + g * LT, LT)], wsem)
                return carry
            lax.fori_loop(0, CHUNK // LT, fire_out, 0)

        def wait_out_f(g, carry):
            pltpu.make_async_copy(out_v.at[pl.ds(0, LT)],
                                  out_hbm.at[cix, pl.ds(0, LT)], wsem).wait()
            return carry
        lax.fori_loop(0, CHUNK // LT, wait_out_f, 0)

    for q in range(CPW):
        do_column(wid * CPW + q)


_sc_call = functools.partial(
    pl.kernel,
    out_type=jax.ShapeDtypeStruct((EMB, N_OUT), jnp.float32),
    mesh=plsc.VectorSubcoreMesh(core_axis_name="c", subcore_axis_name="s"),
    compiler_params=pltpu.CompilerParams(needs_layout_passes=False,
                                         use_tc_tiling_on_sc=True),
    scratch_types=[
        pltpu.VMEM((N_BASE,), jnp.int32),
        pltpu.VMEM((COLS,), jnp.float32),
        pltpu.VMEM((CHUNK,), jnp.int32),
        pltpu.VMEM((CHUNK,), jnp.float32),
        pltpu.SemaphoreType.DMA,
        pltpu.SemaphoreType.DMA,
    ],
)(_sc_body)


def kernel(base_model_tokens, positional_tokens, base_idxs_of_tokens, table):
    pe_t = _posenc(positional_tokens.astype(jnp.int32))
    tail = jnp.pad(table[VFULL * LT:].T, ((0, 0), (0, VPAD - VOCAB)))
    out_t = _sc_call(base_model_tokens.astype(jnp.int32),
                     base_idxs_of_tokens.astype(jnp.int32), pe_t, table.T,
                     tail)
    return out_t.T


# speedup vs baseline: 1.4975x; 1.3055x over previous
import functools
import math

import jax
import jax.numpy as jnp
from jax import lax
from jax.experimental import pallas as pl
from jax.experimental.pallas import tpu as pltpu
from jax.experimental.pallas import tpu_sc as plsc

VOCAB = 100000
EMB = 64
N_BASE = 16384
N_POS = 8192
N_OUT = N_BASE + N_POS

NC, NS, L = 2, 16, 16
NW = NC * NS
CPW = EMB // NW
CHUNK = 1536
NCHUNK = N_OUT // CHUNK
LT = 128
VFULL = VOCAB // LT
VPAD = (VFULL + 1) * LT
COLS = VPAD + N_POS


def _posenc_body(pt_ref, out_ref):
    pt = pt_ref[...].astype(jnp.float32)[None, :]
    row = lax.broadcasted_iota(jnp.int32, (EMB, 1), 0)
    k = (row % (EMB // 2)).astype(jnp.float32)
    period = jnp.exp(k * (-2.0 * math.log(10000.0) / EMB))
    shift = jnp.where(row < EMB // 2, 0.0, 0.5 * math.pi)
    out_ref[...] = jnp.sin(pt * period + shift)


_posenc = pl.pallas_call(
    _posenc_body,
    out_shape=jax.ShapeDtypeStruct((EMB, N_POS), jnp.float32),
)


GU = 6


def _sc_body(bmt_hbm, idx_hbm, pe_hbm, tt_hbm, tail_hbm, out_hbm,
             bmt_v, col_v, idx_v0, idx_v1, out_v0, out_v1,
             lsem, isem, wsem):
    wid = lax.axis_index("s") * NC + lax.axis_index("c")
    idx_bufs = [idx_v0, idx_v1]
    out_bufs = [out_v0, out_v1]
    pltpu.sync_copy(bmt_hbm, bmt_v)

    def do_column(cix):
        def fire_tt(g, carry):
            o = g * (4 * LT)
            for u in range(4):
                pltpu.async_copy(tt_hbm.at[cix, pl.ds(o + u * LT, LT)],
                                 col_v.at[pl.ds(o + u * LT, LT)], lsem)
            return carry
        lax.fori_loop(0, VFULL // 4, fire_tt, 0)
        pltpu.async_copy(tt_hbm.at[cix, pl.ds((VFULL - 1) * LT, LT)],
                         col_v.at[pl.ds((VFULL - 1) * LT, LT)], lsem)
        pltpu.async_copy(tail_hbm.at[cix], col_v.at[pl.ds(VFULL * LT, LT)],
                         lsem)

        def fire_pe(g, carry):
            o = g * (4 * LT)
            for u in range(4):
                pltpu.async_copy(
                    pe_hbm.at[cix, pl.ds(o + u * LT, LT)],
                    col_v.at[pl.ds(VPAD + o + u * LT, LT)], lsem)
            return carry
        lax.fori_loop(0, N_POS // (4 * LT), fire_pe, 0)
        pltpu.async_copy(idx_hbm.at[pl.ds(0, CHUNK)], idx_bufs[0], isem)
        for _ in range(4):
            pltpu.make_async_copy(out_hbm.at[cix],
                                  col_v.at[pl.ds(0, N_OUT)], lsem).wait()
        pltpu.make_async_copy(out_hbm.at[cix, pl.ds(0, COLS - 4 * N_OUT)],
                              col_v.at[pl.ds(0, COLS - 4 * N_OUT)],
                              lsem).wait()

        for ch in range(NCHUNK):
            idx_v = idx_bufs[ch % 2]
            out_v = out_bufs[ch % 2]
            pltpu.make_async_copy(idx_hbm.at[pl.ds(0, CHUNK)], idx_v,
                                  isem).wait()
            if ch + 1 < NCHUNK:
                pltpu.async_copy(
                    idx_hbm.at[pl.ds((ch + 1) * CHUNK, CHUNK)],
                    idx_bufs[(ch + 1) % 2], isem)
            if ch >= 2:
                pltpu.make_async_copy(out_v, out_hbm.at[cix, pl.ds(0, CHUNK)],
                                      wsem).wait()

            def grp(g, carry):
                base = g * (GU * L)
                for u in range(GU):
                    sl = pl.ds(base + u * L, L)
                    iv = idx_v[sl]
                    isb = iv < N_BASE
                    tok = plsc.load_gather(bmt_v, [lax.rem(iv, N_BASE)])
                    comb = jnp.where(isb, tok, iv + (VPAD - N_BASE))
                    out_v[sl] = plsc.load_gather(col_v, [comb])
                return carry
            lax.fori_loop(0, CHUNK // (GU * L), grp, 0)

            def fire_out(g, carry):
                o = g * (4 * LT)
                for u in range(4):
                    pltpu.async_copy(
                        out_v.at[pl.ds(o + u * LT, LT)],
                        out_hbm.at[cix, pl.ds(ch * CHUNK + o + u * LT, LT)],
                        wsem)
                return carry
            lax.fori_loop(0, CHUNK // (4 * LT), fire_out, 0)

        for b in range(2):
            pltpu.make_async_copy(out_bufs[b],
                                  out_hbm.at[cix, pl.ds(0, CHUNK)],
                                  wsem).wait()

    for q in range(CPW):
        do_column(wid * CPW + q)


_sc_call = functools.partial(
    pl.kernel,
    out_type=jax.ShapeDtypeStruct((EMB, N_OUT), jnp.float32),
    mesh=plsc.VectorSubcoreMesh(core_axis_name="c", subcore_axis_name="s"),
    compiler_params=pltpu.CompilerParams(needs_layout_passes=False,
                                         use_tc_tiling_on_sc=True),
    scratch_types=[
        pltpu.VMEM((N_BASE,), jnp.int32),
        pltpu.VMEM((COLS,), jnp.float32),
        pltpu.VMEM((CHUNK,), jnp.int32),
        pltpu.VMEM((CHUNK,), jnp.int32),
        pltpu.VMEM((CHUNK,), jnp.float32),
        pltpu.VMEM((CHUNK,), jnp.float32),
        pltpu.SemaphoreType.DMA,
        pltpu.SemaphoreType.DMA,
        pltpu.SemaphoreType.DMA,
    ],
)(_sc_body)


def kernel(base_model_tokens, positional_tokens, base_idxs_of_tokens, table):
    pe_t = _posenc(positional_tokens.astype(jnp.int32))
    tail = jnp.pad(table[VFULL * LT:].T, ((0, 0), (0, VPAD - VOCAB)))
    out_t = _sc_call(base_model_tokens.astype(jnp.int32),
                     base_idxs_of_tokens.astype(jnp.int32), pe_t, table.T,
                     tail)
    return out_t.T
